# baseline (speedup 1.0000x reference)
import functools

import jax
import jax.numpy as jnp
from jax import lax
from jax.experimental import pallas as pl
from jax.experimental.pallas import tpu as pltpu
from jax.experimental.pallas import tpu_sc as plsc

_N = 10000
_E = 320000
_DIM = 256
_HALF = 128
_NC = 2
_NS = 16
_ACC = 10240
_CH = 128
_GRP = 16
_NG = 10
_NCH = _GRP * _NG
_EPT = _NCH * _CH
_EPAD = _NS * _EPT
_DEG_PT = _E // (_NC * _NS)

_mesh = plsc.VectorSubcoreMesh(core_axis_name="c", subcore_axis_name="s")


@functools.partial(
    pl.kernel,
    out_type=jax.ShapeDtypeStruct((_NC * _NS, _ACC), jnp.float32),
    mesh=_mesh,
    scratch_types=[
        pltpu.VMEM((_DEG_PT,), jnp.int32),
        pltpu.VMEM((_ACC,), jnp.float32),
    ],
    compiler_params=pltpu.CompilerParams(needs_layout_passes=False),
)
def _deg_hist(dst_hbm, out_hbm, dstbuf, hist):
    c = lax.axis_index("c")
    s = lax.axis_index("s")
    wid = s * _NC + c
    pltpu.sync_copy(dst_hbm.at[wid], dstbuf)

    def _zero(i, _):
        hist[pl.ds(i * 16, 16)] = jnp.zeros((16,), jnp.float32)
        return 0

    lax.fori_loop(0, _ACC // 16, _zero, 0)
    ones = jnp.ones((16,), jnp.float32)

    def _step(i, _):
        idx = dstbuf[pl.ds(i * 16, 16)]
        plsc.addupdate_scatter(hist, [idx], ones)
        return 0

    lax.fori_loop(0, _DEG_PT // 16, _step, 0)
    pltpu.sync_copy(hist, out_hbm.at[wid])


@functools.partial(
    pl.kernel,
    out_type=jax.ShapeDtypeStruct((_NC * _N, _HALF), jnp.float32),
    mesh=_mesh,
    scratch_types=[
        pltpu.VMEM((2, _GRP, 1, _CH), jnp.int32),
        pltpu.VMEM((2, _GRP, 1, _CH), jnp.int32),
        pltpu.VMEM((_CH, _HALF), jnp.float32),
        pltpu.VMEM((_CH, _HALF), jnp.float32),
        pltpu.VMEM_SHARED((_ACC, _HALF), jnp.float32),
        pltpu.SemaphoreType.DMA,
        pltpu.SemaphoreType.DMA,
        pltpu.SemaphoreType.DMA,
        pltpu.SemaphoreType.DMA,
        pltpu.SemaphoreType.DMA,
        pltpu.SemaphoreType.DMA,
        pltpu.SemaphoreType.DMA,
    ],
    compiler_params=pltpu.CompilerParams(needs_layout_passes=False),
)
def _edge_agg(h2_hbm, srci_hbm, dsti_hbm, out_hbm,
              sidx, didx, rowa, rowb, acc,
              sema0, sema1, semb0, semb1, semsa, semsb, semi):
    c = lax.axis_index("c")
    s = lax.axis_index("s")

    def _zero(i, _):
        rowa[i // 8, pl.ds((i % 8) * 16, 16)] = jnp.zeros((16,), jnp.float32)
        return 0

    lax.fori_loop(0, _CH * 8, _zero, 0)
    for k in range(_ACC // _NS // _CH):
        pltpu.sync_copy(rowa, acc.at[pl.ds(s * (_ACC // _NS) + k * _CH, _CH)])
    plsc.subcore_barrier()

    pltpu.async_copy(srci_hbm.at[c, s, 0], sidx.at[0], semi)
    pltpu.async_copy(dsti_hbm.at[s, 0], didx.at[0], semi)

    def _group(g, _):
        p = g % 2
        pltpu.make_async_copy(srci_hbm.at[c, s, g], sidx.at[p], semi).wait()
        pltpu.make_async_copy(dsti_hbm.at[s, g], didx.at[p], semi).wait()

        @pl.when(g + 1 < _NG)
        def _():
            pltpu.async_copy(srci_hbm.at[c, s, g + 1], sidx.at[1 - p], semi)
            pltpu.async_copy(dsti_hbm.at[s, g + 1], didx.at[1 - p], semi)

        def _gissue(k, buf, g0, g1):
            h = _CH // 2
            pltpu.async_copy(h2_hbm.at[sidx.at[p, k, 0, pl.ds(0, h)]],
                             buf.at[pl.ds(0, h)], g0)
            pltpu.async_copy(h2_hbm.at[sidx.at[p, k, 0, pl.ds(h, h)]],
                             buf.at[pl.ds(h, h)], g1)

        def _gwait(k, buf, g0, g1):
            h = _CH // 2
            pltpu.make_async_copy(h2_hbm.at[sidx.at[p, k, 0, pl.ds(0, h)]],
                                  buf.at[pl.ds(0, h)], g0).wait()
            pltpu.make_async_copy(h2_hbm.at[sidx.at[p, k, 0, pl.ds(h, h)]],
                                  buf.at[pl.ds(h, h)], g1).wait()

        _gissue(0, rowa, sema0, sema1)
        for k in range(_GRP):
            if k % 2 == 0:
                buf, g0, g1, ssem = rowa, sema0, sema1, semsa
                nbuf, n0, n1, nssem = rowb, semb0, semb1, semsb
            else:
                buf, g0, g1, ssem = rowb, semb0, semb1, semsb
                nbuf, n0, n1, nssem = rowa, sema0, sema1, semsa
            if k == 0:
                @pl.when(g > 0)
                def _():
                    pltpu.make_async_copy(nbuf, acc.at[didx.at[p, 0, 0]],
                                          nssem).wait()
            else:
                pltpu.make_async_copy(nbuf, acc.at[didx.at[p, k, 0]],
                                      nssem).wait()
            if k + 1 < _GRP:
                _gissue(k + 1, nbuf, n0, n1)
            _gwait(k, buf, g0, g1)
            pltpu.async_copy(buf, acc.at[didx.at[p, k, 0]], ssem, add=True)
        return 0

    lax.fori_loop(0, _NG, _group, 0)
    pltpu.make_async_copy(rowb, acc.at[didx.at[(_NG - 1) % 2, _GRP - 1, 0]],
                          semsb).wait()
    plsc.subcore_barrier()
    @pl.when(s < _NS - 1)
    def _():
        pltpu.sync_copy(acc.at[pl.ds(s * 632, 632)],
                        out_hbm.at[pl.ds(c * _N + s * 632, 632)])

    @pl.when(s == _NS - 1)
    def _():
        pltpu.sync_copy(acc.at[pl.ds(15 * 632, 520)],
                        out_hbm.at[pl.ds(c * _N + 15 * 632, 520)])


def _k1_body(hists_ref, x_ref, w1_ref, h2_ref, dis_ref):
    ones = jnp.ones((1, _NC * _NS), jnp.float32)
    deg_row = jnp.dot(ones, hists_ref[...]) + 1.0
    dis = jnp.transpose(lax.rsqrt(deg_row))[:_N, :]
    dis_ref[...] = dis
    h2 = jnp.dot(x_ref[...], w1_ref[...],
                 preferred_element_type=jnp.float32) * dis
    h2_ref[pl.ds(0, _N), :] = h2[:, :_HALF]
    h2_ref[pl.ds(_N, _N), :] = h2[:, _HALF:]


def _tc_k1(hists, x, w1):
    return pl.pallas_call(
        _k1_body,
        out_shape=(jax.ShapeDtypeStruct((_NC * _N, _HALF), jnp.float32),
                   jax.ShapeDtypeStruct((_N, 1), jnp.float32)),
    )(hists, x, w1)


def _mid_body(s_ref, h2_ref, dis_ref, b_ref, g_ref, beta_ref, w_ref,
              out_ref, *, relu):
    dis = dis_ref[...]
    t = jnp.concatenate(
        [s_ref[pl.ds(0, _N), :] + h2_ref[pl.ds(0, _N), :],
         s_ref[pl.ds(_N, _N), :] + h2_ref[pl.ds(_N, _N), :]], axis=1)
    t = dis * t + b_ref[...]
    if relu:
        t = jnp.maximum(t, 0.0)
    mu = jnp.mean(t, axis=0, keepdims=True)
    var = jnp.mean((t - mu) ** 2, axis=0, keepdims=True)
    hb = g_ref[...] * (t - mu) * lax.rsqrt(var + 1e-5) + beta_ref[...]
    h2n = jnp.dot(hb, w_ref[...], preferred_element_type=jnp.float32) * dis
    out_ref[pl.ds(0, _N), :] = h2n[:, :_HALF]
    out_ref[pl.ds(_N, _N), :] = h2n[:, _HALF:]


def _tc_mid(s, h2, dis, b, g, beta, w, relu):
    return pl.pallas_call(
        functools.partial(_mid_body, relu=relu),
        out_shape=jax.ShapeDtypeStruct((_NC * _N, _HALF), jnp.float32),
    )(s, h2, dis, b.reshape(1, _DIM), g.reshape(1, _DIM),
      beta.reshape(1, _DIM), w)


def _final_body(s_ref, h2_ref, dis_ref, b_ref, g_ref, beta_ref, batch_ref,
                wm_ref, bm_ref, gm_ref, betam_ref, wo_ref, bo_ref, out_ref):
    dis = dis_ref[...]
    t = jnp.concatenate(
        [s_ref[pl.ds(0, _N), :] + h2_ref[pl.ds(0, _N), :],
         s_ref[pl.ds(_N, _N), :] + h2_ref[pl.ds(_N, _N), :]], axis=1)
    t = dis * t + b_ref[...]
    mu = jnp.mean(t, axis=0, keepdims=True)
    var = jnp.mean((t - mu) ** 2, axis=0, keepdims=True)
    hb = g_ref[...] * (t - mu) * lax.rsqrt(var + 1e-5) + beta_ref[...]
    seg = lax.broadcasted_iota(jnp.int32, (64, _N), 0)
    onehot = (seg == batch_ref[...]).astype(jnp.float32)
    p = jnp.dot(onehot, hb, preferred_element_type=jnp.float32)
    for i in range(2):
        p = jnp.maximum(jnp.dot(p, wm_ref[i], preferred_element_type=jnp.float32)
                        + bm_ref[pl.ds(i, 1), :], 0.0)
        mu = jnp.mean(p, axis=0, keepdims=True)
        var = jnp.mean((p - mu) ** 2, axis=0, keepdims=True)
        p = (gm_ref[pl.ds(i, 1), :] * (p - mu) * lax.rsqrt(var + 1e-5)
             + betam_ref[pl.ds(i, 1), :])
    out_ref[...] = jnp.dot(p, wo_ref[...],
                           preferred_element_type=jnp.float32) + bo_ref[...]


def _tc_final(s, h2, dis, b, g, beta, batch, wm, bm, gm, betam, wo, bo):
    return pl.pallas_call(
        _final_body,
        out_shape=jax.ShapeDtypeStruct((64, 1), jnp.float32),
    )(s, h2, dis, b.reshape(1, _DIM), g.reshape(1, _DIM),
      beta.reshape(1, _DIM), batch.reshape(1, _N), wm, bm, gm, betam,
      wo, bo.reshape(1, 1))


def kernel(x, edge_index, batch, W1, b1, g1, beta1, Wh, bh, gh, betah,
           Wm, bm, gm, betam, Wo, bo):
    src = edge_index[0]
    dst = edge_index[1]
    pad = _EPAD - _E
    srcp = jnp.concatenate([src, jnp.zeros((pad,), jnp.int32)])
    dstp = jnp.concatenate([dst, jnp.full((pad,), _N + 8, jnp.int32)])
    srct = srcp.reshape(_NS, _NG, _GRP, 1, _CH)
    srci = jnp.stack([srct, srct + _N])
    dsti = dstp.reshape(_NS, _NG, _GRP, 1, _CH)
    dst32 = dst.reshape(_NC * _NS, _DEG_PT)

    hists = _deg_hist(dst32)
    h2, dis = _tc_k1(hists, x, W1)
    s = _edge_agg(h2, srci, dsti)
    h2 = _tc_mid(s, h2, dis, b1, g1, beta1, Wh[0], relu=True)
    s = _edge_agg(h2, srci, dsti)
    h2 = _tc_mid(s, h2, dis, bh[0], gh[0], betah[0], Wh[1], relu=False)
    s = _edge_agg(h2, srci, dsti)
    h2 = _tc_mid(s, h2, dis, bh[1], gh[1], betah[1], Wh[2], relu=False)
    s = _edge_agg(h2, srci, dsti)
    return _tc_final(s, h2, dis, bh[2], gh[2], betah[2], batch,
                     Wm, bm, gm, betam, Wo, bo)


# speedup vs baseline: 1.0067x; 1.0067x over previous
import functools

import jax
import jax.numpy as jnp
from jax import lax
from jax.experimental import pallas as pl
from jax.experimental.pallas import tpu as pltpu
from jax.experimental.pallas import tpu_sc as plsc

_N = 10000
_E = 320000
_DIM = 256
_HALF = 128
_NC = 2
_NS = 16
_ACC = 10240
_CH = 128
_GRP = 16
_NG = 10
_NCH = _GRP * _NG
_EPT = _NCH * _CH
_EPAD = _NS * _EPT
_DEG_PT = _E // (_NC * _NS)

_mesh = plsc.VectorSubcoreMesh(core_axis_name="c", subcore_axis_name="s")


@functools.partial(
    pl.kernel,
    out_type=jax.ShapeDtypeStruct((_NC * _NS, _ACC), jnp.float32),
    mesh=_mesh,
    scratch_types=[
        pltpu.VMEM((_DEG_PT,), jnp.int32),
        pltpu.VMEM((_ACC,), jnp.float32),
    ],
    compiler_params=pltpu.CompilerParams(needs_layout_passes=False),
)
def _deg_hist(dst_hbm, out_hbm, dstbuf, hist):
    c = lax.axis_index("c")
    s = lax.axis_index("s")
    wid = s * _NC + c
    pltpu.sync_copy(dst_hbm.at[wid], dstbuf)

    def _zero(i, _):
        hist[pl.ds(i * 16, 16)] = jnp.zeros((16,), jnp.float32)
        return 0

    lax.fori_loop(0, _ACC // 16, _zero, 0)
    ones = jnp.ones((16,), jnp.float32)

    def _step(i, _):
        idx = dstbuf[pl.ds(i * 16, 16)]
        plsc.addupdate_scatter(hist, [idx], ones)
        return 0

    lax.fori_loop(0, _DEG_PT // 16, _step, 0)
    pltpu.sync_copy(hist, out_hbm.at[wid])


@functools.partial(
    pl.kernel,
    out_type=jax.ShapeDtypeStruct((_NC * _N, _HALF), jnp.float32),
    mesh=_mesh,
    scratch_types=[
        pltpu.VMEM((2, _GRP, 1, _CH), jnp.int32),
        pltpu.VMEM((2, _GRP, 1, _CH), jnp.int32),
        pltpu.VMEM((_CH, _HALF), jnp.float32),
        pltpu.VMEM((_CH, _HALF), jnp.float32),
        pltpu.VMEM_SHARED((_ACC, _HALF), jnp.float32),
        pltpu.SemaphoreType.DMA,
        pltpu.SemaphoreType.DMA,
        pltpu.SemaphoreType.DMA,
        pltpu.SemaphoreType.DMA,
        pltpu.SemaphoreType.DMA,
    ],
    compiler_params=pltpu.CompilerParams(needs_layout_passes=False),
)
def _edge_agg(h2_hbm, srci_hbm, dsti_hbm, out_hbm,
              sidx, didx, rowa, rowb, acc,
              sema, semb, semsa, semsb, semi):
    c = lax.axis_index("c")
    s = lax.axis_index("s")

    def _zero(i, _):
        rowa[i // 8, pl.ds((i % 8) * 16, 16)] = jnp.zeros((16,), jnp.float32)
        return 0

    lax.fori_loop(0, _CH * 8, _zero, 0)
    for k in range(_ACC // _NS // _CH):
        pltpu.sync_copy(rowa, acc.at[pl.ds(s * (_ACC // _NS) + k * _CH, _CH)])
    plsc.subcore_barrier()

    pltpu.async_copy(srci_hbm.at[c, s, 0], sidx.at[0], semi)
    pltpu.async_copy(dsti_hbm.at[s, 0], didx.at[0], semi)
    pltpu.make_async_copy(srci_hbm.at[c, s, 0], sidx.at[0], semi).wait()
    pltpu.make_async_copy(dsti_hbm.at[s, 0], didx.at[0], semi).wait()
    pltpu.async_copy(srci_hbm.at[c, s, 1], sidx.at[1], semi)
    pltpu.async_copy(dsti_hbm.at[s, 1], didx.at[1], semi)
    pltpu.async_copy(h2_hbm.at[sidx.at[0, 0, 0]], rowa, sema)

    def _group(g, _):
        p = g % 2
        for k in range(_GRP):
            buf, gsem, ssem = (rowa, sema, semsa) if k % 2 == 0 else (rowb, semb, semsb)
            nbuf, ngsem, nssem = (rowb, semb, semsb) if k % 2 == 0 else (rowa, sema, semsa)
            if k == 0:
                @pl.when(g > 0)
                def _():
                    pltpu.make_async_copy(nbuf, acc.at[didx.at[p, 0, 0]],
                                          nssem).wait()
            else:
                pltpu.make_async_copy(nbuf, acc.at[didx.at[p, k, 0]],
                                      nssem).wait()
            if k + 1 < _GRP:
                pltpu.async_copy(h2_hbm.at[sidx.at[p, k + 1, 0]], nbuf, ngsem)
            pltpu.make_async_copy(h2_hbm.at[sidx.at[p, k, 0]], buf, gsem).wait()
            pltpu.async_copy(buf, acc.at[didx.at[p, k, 0]], ssem, add=True)

        @pl.when(g + 1 < _NG)
        def _():
            pltpu.make_async_copy(srci_hbm.at[c, s, g + 1], sidx.at[1 - p],
                                  semi).wait()
            pltpu.make_async_copy(dsti_hbm.at[s, g + 1], didx.at[1 - p],
                                  semi).wait()

            @pl.when(g + 2 < _NG)
            def _():
                pltpu.async_copy(srci_hbm.at[c, s, g + 2], sidx.at[p], semi)
                pltpu.async_copy(dsti_hbm.at[s, g + 2], didx.at[p], semi)

            pltpu.async_copy(h2_hbm.at[sidx.at[1 - p, 0, 0]], rowa, sema)
        return 0

    lax.fori_loop(0, _NG, _group, 0)
    pltpu.make_async_copy(rowb, acc.at[didx.at[(_NG - 1) % 2, _GRP - 1, 0]],
                          semsb).wait()
    plsc.subcore_barrier()
    @pl.when(s < _NS - 1)
    def _():
        pltpu.sync_copy(acc.at[pl.ds(s * 632, 632)],
                        out_hbm.at[pl.ds(c * _N + s * 632, 632)])

    @pl.when(s == _NS - 1)
    def _():
        pltpu.sync_copy(acc.at[pl.ds(15 * 632, 520)],
                        out_hbm.at[pl.ds(c * _N + 15 * 632, 520)])


def _k1_body(hists_ref, x_ref, w1_ref, h2_ref, dis_ref):
    ones = jnp.ones((1, _NC * _NS), jnp.float32)
    deg_row = jnp.dot(ones, hists_ref[...]) + 1.0
    dis = jnp.transpose(lax.rsqrt(deg_row))[:_N, :]
    dis_ref[...] = dis
    h2 = jnp.dot(x_ref[...], w1_ref[...],
                 preferred_element_type=jnp.float32) * dis
    h2_ref[pl.ds(0, _N), :] = h2[:, :_HALF]
    h2_ref[pl.ds(_N, _N), :] = h2[:, _HALF:]


def _tc_k1(hists, x, w1):
    return pl.pallas_call(
        _k1_body,
        out_shape=(jax.ShapeDtypeStruct((_NC * _N, _HALF), jnp.float32),
                   jax.ShapeDtypeStruct((_N, 1), jnp.float32)),
    )(hists, x, w1)


def _mid_body(s_ref, h2_ref, dis_ref, b_ref, g_ref, beta_ref, w_ref,
              out_ref, *, relu):
    dis = dis_ref[...]
    t = jnp.concatenate(
        [s_ref[pl.ds(0, _N), :] + h2_ref[pl.ds(0, _N), :],
         s_ref[pl.ds(_N, _N), :] + h2_ref[pl.ds(_N, _N), :]], axis=1)
    t = dis * t + b_ref[...]
    if relu:
        t = jnp.maximum(t, 0.0)
    mu = jnp.mean(t, axis=0, keepdims=True)
    var = jnp.mean((t - mu) ** 2, axis=0, keepdims=True)
    hb = g_ref[...] * (t - mu) * lax.rsqrt(var + 1e-5) + beta_ref[...]
    h2n = jnp.dot(hb, w_ref[...], preferred_element_type=jnp.float32) * dis
    out_ref[pl.ds(0, _N), :] = h2n[:, :_HALF]
    out_ref[pl.ds(_N, _N), :] = h2n[:, _HALF:]


def _tc_mid(s, h2, dis, b, g, beta, w, relu):
    return pl.pallas_call(
        functools.partial(_mid_body, relu=relu),
        out_shape=jax.ShapeDtypeStruct((_NC * _N, _HALF), jnp.float32),
    )(s, h2, dis, b.reshape(1, _DIM), g.reshape(1, _DIM),
      beta.reshape(1, _DIM), w)


def _final_body(s_ref, h2_ref, dis_ref, b_ref, g_ref, beta_ref, batch_ref,
                wm_ref, bm_ref, gm_ref, betam_ref, wo_ref, bo_ref, out_ref):
    dis = dis_ref[...]
    t = jnp.concatenate(
        [s_ref[pl.ds(0, _N), :] + h2_ref[pl.ds(0, _N), :],
         s_ref[pl.ds(_N, _N), :] + h2_ref[pl.ds(_N, _N), :]], axis=1)
    t = dis * t + b_ref[...]
    mu = jnp.mean(t, axis=0, keepdims=True)
    var = jnp.mean((t - mu) ** 2, axis=0, keepdims=True)
    hb = g_ref[...] * (t - mu) * lax.rsqrt(var + 1e-5) + beta_ref[...]
    seg = lax.broadcasted_iota(jnp.int32, (64, _N), 0)
    onehot = (seg == batch_ref[...]).astype(jnp.float32)
    p = jnp.dot(onehot, hb, preferred_element_type=jnp.float32)
    for i in range(2):
        p = jnp.maximum(jnp.dot(p, wm_ref[i], preferred_element_type=jnp.float32)
                        + bm_ref[pl.ds(i, 1), :], 0.0)
        mu = jnp.mean(p, axis=0, keepdims=True)
        var = jnp.mean((p - mu) ** 2, axis=0, keepdims=True)
        p = (gm_ref[pl.ds(i, 1), :] * (p - mu) * lax.rsqrt(var + 1e-5)
             + betam_ref[pl.ds(i, 1), :])
    out_ref[...] = jnp.dot(p, wo_ref[...],
                           preferred_element_type=jnp.float32) + bo_ref[...]


def _tc_final(s, h2, dis, b, g, beta, batch, wm, bm, gm, betam, wo, bo):
    return pl.pallas_call(
        _final_body,
        out_shape=jax.ShapeDtypeStruct((64, 1), jnp.float32),
    )(s, h2, dis, b.reshape(1, _DIM), g.reshape(1, _DIM),
      beta.reshape(1, _DIM), batch.reshape(1, _N), wm, bm, gm, betam,
      wo, bo.reshape(1, 1))


def kernel(x, edge_index, batch, W1, b1, g1, beta1, Wh, bh, gh, betah,
           Wm, bm, gm, betam, Wo, bo):
    src = edge_index[0]
    dst = edge_index[1]
    pad = _EPAD - _E
    srcp = jnp.concatenate([src, jnp.zeros((pad,), jnp.int32)])
    dstp = jnp.concatenate([dst, jnp.full((pad,), _N + 8, jnp.int32)])
    srct = srcp.reshape(_NS, _NG, _GRP, 1, _CH)
    srci = jnp.stack([srct, srct + _N])
    dsti = dstp.reshape(_NS, _NG, _GRP, 1, _CH)
    dst32 = dst.reshape(_NC * _NS, _DEG_PT)

    hists = _deg_hist(dst32)
    h2, dis = _tc_k1(hists, x, W1)
    s = _edge_agg(h2, srci, dsti)
    h2 = _tc_mid(s, h2, dis, b1, g1, beta1, Wh[0], relu=True)
    s = _edge_agg(h2, srci, dsti)
    h2 = _tc_mid(s, h2, dis, bh[0], gh[0], betah[0], Wh[1], relu=False)
    s = _edge_agg(h2, srci, dsti)
    h2 = _tc_mid(s, h2, dis, bh[1], gh[1], betah[1], Wh[2], relu=False)
    s = _edge_agg(h2, srci, dsti)
    return _tc_final(s, h2, dis, bh[2], gh[2], betah[2], batch,
                     Wm, bm, gm, betam, Wo, bo)


# speedup vs baseline: 1.0084x; 1.0017x over previous
import functools

import jax
import jax.numpy as jnp
from jax import lax
from jax.experimental import pallas as pl
from jax.experimental.pallas import tpu as pltpu
from jax.experimental.pallas import tpu_sc as plsc

_N = 10000
_E = 320000
_DIM = 256
_HALF = 128
_NC = 2
_NS = 16
_ACC = 10240
_CH = 128
_GRP = 16
_NG = 10
_NCH = _GRP * _NG
_EPT = _NCH * _CH
_EPAD = _NS * _EPT
_DEG_PT = _E // (_NC * _NS)

_mesh = plsc.VectorSubcoreMesh(core_axis_name="c", subcore_axis_name="s")


@functools.partial(
    pl.kernel,
    out_type=jax.ShapeDtypeStruct((_NC * _NS, _ACC), jnp.float32),
    mesh=_mesh,
    scratch_types=[
        pltpu.VMEM((_DEG_PT,), jnp.int32),
        pltpu.VMEM((_ACC,), jnp.float32),
    ],
    compiler_params=pltpu.CompilerParams(needs_layout_passes=False),
)
def _deg_hist(dst_hbm, out_hbm, dstbuf, hist):
    c = lax.axis_index("c")
    s = lax.axis_index("s")
    wid = s * _NC + c
    pltpu.sync_copy(dst_hbm.at[wid], dstbuf)

    def _zero(i, _):
        hist[pl.ds(i * 16, 16)] = jnp.zeros((16,), jnp.float32)
        return 0

    lax.fori_loop(0, _ACC // 16, _zero, 0)
    ones = jnp.ones((16,), jnp.float32)

    def _step(i, _):
        idx = dstbuf[pl.ds(i * 16, 16)]
        plsc.addupdate_scatter(hist, [idx], ones)
        return 0

    lax.fori_loop(0, _DEG_PT // 16, _step, 0)
    pltpu.sync_copy(hist, out_hbm.at[wid])


@functools.partial(
    pl.kernel,
    out_type=jax.ShapeDtypeStruct((_NC * _N, _HALF), jnp.float32),
    mesh=_mesh,
    scratch_types=[
        pltpu.VMEM((2, _GRP, 1, _CH), jnp.int32),
        pltpu.VMEM((2, _GRP, 1, _CH), jnp.int32),
        pltpu.VMEM((_CH, _HALF), jnp.float32),
        pltpu.VMEM((_CH, _HALF), jnp.float32),
        pltpu.VMEM_SHARED((_ACC, _HALF), jnp.float32),
        pltpu.SemaphoreType.DMA,
        pltpu.SemaphoreType.DMA,
        pltpu.SemaphoreType.DMA,
        pltpu.SemaphoreType.DMA,
        pltpu.SemaphoreType.DMA,
    ],
    compiler_params=pltpu.CompilerParams(needs_layout_passes=False),
)
def _edge_agg(h2_hbm, srci_hbm, dsti_hbm, zer_hbm, out_hbm,
              sidx, didx, rowa, rowb, acc,
              sema, semb, semsa, semsb, semi):
    c = lax.axis_index("c")
    s = lax.axis_index("s")

    pltpu.async_copy(srci_hbm.at[c, s, 0], sidx.at[0], semi)
    pltpu.async_copy(dsti_hbm.at[s, 0], didx.at[0], semi)
    pltpu.async_copy(srci_hbm.at[c, s, 1], sidx.at[1], semi)
    pltpu.async_copy(dsti_hbm.at[s, 1], didx.at[1], semi)

    pltpu.sync_copy(zer_hbm, rowb)
    for k in range(_ACC // _NS // _CH):
        pltpu.sync_copy(rowb, acc.at[pl.ds(s * (_ACC // _NS) + k * _CH, _CH)])
    plsc.subcore_barrier()

    pltpu.make_async_copy(srci_hbm.at[c, s, 0], sidx.at[0], semi).wait()
    pltpu.make_async_copy(dsti_hbm.at[s, 0], didx.at[0], semi).wait()
    pltpu.async_copy(h2_hbm.at[sidx.at[0, 0, 0]], rowa, sema)

    def _group(g, _):
        p = g % 2
        for k in range(_GRP):
            buf, gsem, ssem = (rowa, sema, semsa) if k % 2 == 0 else (rowb, semb, semsb)
            nbuf, ngsem, nssem = (rowb, semb, semsb) if k % 2 == 0 else (rowa, sema, semsa)
            if k == 0:
                @pl.when(g > 0)
                def _():
                    pltpu.make_async_copy(nbuf, acc.at[didx.at[p, 0, 0]],
                                          nssem).wait()
            else:
                pltpu.make_async_copy(nbuf, acc.at[didx.at[p, k, 0]],
                                      nssem).wait()
            if k + 1 < _GRP:
                pltpu.async_copy(h2_hbm.at[sidx.at[p, k + 1, 0]], nbuf, ngsem)
            pltpu.make_async_copy(h2_hbm.at[sidx.at[p, k, 0]], buf, gsem).wait()
            pltpu.async_copy(buf, acc.at[didx.at[p, k, 0]], ssem, add=True)

        @pl.when(g + 1 < _NG)
        def _():
            pltpu.make_async_copy(srci_hbm.at[c, s, g + 1], sidx.at[1 - p],
                                  semi).wait()
            pltpu.make_async_copy(dsti_hbm.at[s, g + 1], didx.at[1 - p],
                                  semi).wait()

            @pl.when(g + 2 < _NG)
            def _():
                pltpu.async_copy(srci_hbm.at[c, s, g + 2], sidx.at[p], semi)
                pltpu.async_copy(dsti_hbm.at[s, g + 2], didx.at[p], semi)

            pltpu.async_copy(h2_hbm.at[sidx.at[1 - p, 0, 0]], rowa, sema)
        return 0

    lax.fori_loop(0, _NG, _group, 0)
    pltpu.make_async_copy(rowb, acc.at[didx.at[(_NG - 1) % 2, _GRP - 1, 0]],
                          semsb).wait()
    plsc.subcore_barrier()
    @pl.when(s < _NS - 1)
    def _():
        pltpu.sync_copy(acc.at[pl.ds(s * 632, 632)],
                        out_hbm.at[pl.ds(c * _N + s * 632, 632)])

    @pl.when(s == _NS - 1)
    def _():
        pltpu.sync_copy(acc.at[pl.ds(15 * 632, 520)],
                        out_hbm.at[pl.ds(c * _N + 15 * 632, 520)])


def _k1_body(hists_ref, x_ref, w1_ref, h2_ref, dis_ref):
    ones = jnp.ones((1, _NC * _NS), jnp.float32)
    deg_row = jnp.dot(ones, hists_ref[...]) + 1.0
    dis = jnp.transpose(lax.rsqrt(deg_row))[:_N, :]
    dis_ref[...] = dis
    h2 = jnp.dot(x_ref[...], w1_ref[...],
                 preferred_element_type=jnp.float32) * dis
    h2_ref[pl.ds(0, _N), :] = h2[:, :_HALF]
    h2_ref[pl.ds(_N, _N), :] = h2[:, _HALF:]


def _tc_k1(hists, x, w1):
    return pl.pallas_call(
        _k1_body,
        out_shape=(jax.ShapeDtypeStruct((_NC * _N, _HALF), jnp.float32),
                   jax.ShapeDtypeStruct((_N, 1), jnp.float32)),
    )(hists, x, w1)


def _mid_body(s_ref, h2_ref, dis_ref, b_ref, g_ref, beta_ref, w_ref,
              out_ref, *, relu):
    dis = dis_ref[...]
    t = jnp.concatenate(
        [s_ref[pl.ds(0, _N), :] + h2_ref[pl.ds(0, _N), :],
         s_ref[pl.ds(_N, _N), :] + h2_ref[pl.ds(_N, _N), :]], axis=1)
    t = dis * t + b_ref[...]
    if relu:
        t = jnp.maximum(t, 0.0)
    mu = jnp.mean(t, axis=0, keepdims=True)
    var = jnp.mean((t - mu) ** 2, axis=0, keepdims=True)
    hb = g_ref[...] * (t - mu) * lax.rsqrt(var + 1e-5) + beta_ref[...]
    h2n = jnp.dot(hb, w_ref[...], preferred_element_type=jnp.float32) * dis
    out_ref[pl.ds(0, _N), :] = h2n[:, :_HALF]
    out_ref[pl.ds(_N, _N), :] = h2n[:, _HALF:]


def _tc_mid(s, h2, dis, b, g, beta, w, relu):
    return pl.pallas_call(
        functools.partial(_mid_body, relu=relu),
        out_shape=jax.ShapeDtypeStruct((_NC * _N, _HALF), jnp.float32),
    )(s, h2, dis, b.reshape(1, _DIM), g.reshape(1, _DIM),
      beta.reshape(1, _DIM), w)


def _final_body(s_ref, h2_ref, dis_ref, b_ref, g_ref, beta_ref, batch_ref,
                wm_ref, bm_ref, gm_ref, betam_ref, wo_ref, bo_ref, out_ref):
    dis = dis_ref[...]
    t = jnp.concatenate(
        [s_ref[pl.ds(0, _N), :] + h2_ref[pl.ds(0, _N), :],
         s_ref[pl.ds(_N, _N), :] + h2_ref[pl.ds(_N, _N), :]], axis=1)
    t = dis * t + b_ref[...]
    mu = jnp.mean(t, axis=0, keepdims=True)
    var = jnp.mean((t - mu) ** 2, axis=0, keepdims=True)
    hb = g_ref[...] * (t - mu) * lax.rsqrt(var + 1e-5) + beta_ref[...]
    seg = lax.broadcasted_iota(jnp.int32, (64, _N), 0)
    onehot = (seg == batch_ref[...]).astype(jnp.float32)
    p = jnp.dot(onehot, hb, preferred_element_type=jnp.float32)
    for i in range(2):
        p = jnp.maximum(jnp.dot(p, wm_ref[i], preferred_element_type=jnp.float32)
                        + bm_ref[pl.ds(i, 1), :], 0.0)
        mu = jnp.mean(p, axis=0, keepdims=True)
        var = jnp.mean((p - mu) ** 2, axis=0, keepdims=True)
        p = (gm_ref[pl.ds(i, 1), :] * (p - mu) * lax.rsqrt(var + 1e-5)
             + betam_ref[pl.ds(i, 1), :])
    out_ref[...] = jnp.dot(p, wo_ref[...],
                           preferred_element_type=jnp.float32) + bo_ref[...]


def _tc_final(s, h2, dis, b, g, beta, batch, wm, bm, gm, betam, wo, bo):
    return pl.pallas_call(
        _final_body,
        out_shape=jax.ShapeDtypeStruct((64, 1), jnp.float32),
    )(s, h2, dis, b.reshape(1, _DIM), g.reshape(1, _DIM),
      beta.reshape(1, _DIM), batch.reshape(1, _N), wm, bm, gm, betam,
      wo, bo.reshape(1, 1))


def kernel(x, edge_index, batch, W1, b1, g1, beta1, Wh, bh, gh, betah,
           Wm, bm, gm, betam, Wo, bo):
    src = edge_index[0]
    dst = edge_index[1]
    pad = _EPAD - _E
    srcp = jnp.concatenate([src, jnp.zeros((pad,), jnp.int32)])
    dstp = jnp.concatenate([dst, jnp.full((pad,), _N + 8, jnp.int32)])
    srct = srcp.reshape(_NS, _NG, _GRP, 1, _CH)
    srci = jnp.stack([srct, srct + _N])
    dsti = dstp.reshape(_NS, _NG, _GRP, 1, _CH)
    dst32 = dst.reshape(_NC * _NS, _DEG_PT)
    zer = jnp.zeros((_CH, _HALF), jnp.float32)

    hists = _deg_hist(dst32)
    h2, dis = _tc_k1(hists, x, W1)
    s = _edge_agg(h2, srci, dsti, zer)
    h2 = _tc_mid(s, h2, dis, b1, g1, beta1, Wh[0], relu=True)
    s = _edge_agg(h2, srci, dsti, zer)
    h2 = _tc_mid(s, h2, dis, bh[0], gh[0], betah[0], Wh[1], relu=False)
    s = _edge_agg(h2, srci, dsti, zer)
    h2 = _tc_mid(s, h2, dis, bh[1], gh[1], betah[1], Wh[2], relu=False)
    s = _edge_agg(h2, srci, dsti, zer)
    return _tc_final(s, h2, dis, bh[2], gh[2], betah[2], batch,
                     Wm, bm, gm, betam, Wo, bo)
